# baseline (speedup 1.0000x reference)
import functools

import jax
import jax.numpy as jnp
from jax import lax
from jax.experimental import pallas as pl
from jax.experimental.pallas import tpu as pltpu
from jax.experimental.pallas import tpu_sc as plsc

VOCAB = 100000
DIM = 64
DPAIR = 128
B = 1024
MAX_NORM = 1.0

_NC = 2
_NS = 16
_NW = _NC * _NS
_B_PER_W = B // _NW

_N_BLK = 2048
_N_GRID = (VOCAB + _N_BLK - 1) // _N_BLK


@functools.lru_cache(maxsize=1)
def _make_sc_gather():
    @functools.partial(
        pl.kernel,
        mesh=plsc.VectorSubcoreMesh(core_axis_name="c", subcore_axis_name="s"),
        out_type=jax.ShapeDtypeStruct((B, DPAIR), jnp.float32),
        scratch_types=[
            pltpu.VMEM((_B_PER_W,), jnp.int32),
            pltpu.VMEM((_B_PER_W, DPAIR), jnp.float32),
            pltpu.SemaphoreType.DMA,
        ],
    )
    def _sc_gather(idx_hbm, table_hbm, out_hbm, idx_v, rows_v, sem):
        wid = lax.axis_index("s") * _NC + lax.axis_index("c")
        base = wid * _B_PER_W
        pltpu.sync_copy(idx_hbm.at[pl.ds(base, _B_PER_W)], idx_v)
        pltpu.async_copy(table_hbm.at[idx_v], rows_v, sem).wait()
        pltpu.sync_copy(rows_v, out_hbm.at[pl.ds(base, _B_PER_W)])

    return _sc_gather


def _mm_body(emb2_ref, par_ref, wt_ref, b_ref, out_ref):
    lo = emb2_ref[:, :DIM]
    hi = emb2_ref[:, DIM:]
    emb = jnp.where(par_ref[...] > 0, hi, lo)
    ss = jnp.sum(emb * emb, axis=1, keepdims=True)
    scale = jnp.where(
        ss > MAX_NORM * MAX_NORM, MAX_NORM / (jnp.sqrt(ss) + 1e-7), 1.0
    )
    emb = emb * scale
    acc = lax.dot_general(
        wt_ref[...],
        emb,
        (((0,), (1,)), ((), ())),
        preferred_element_type=jnp.float32,
    )
    ones = jnp.ones((1, B), dtype=jnp.float32)
    bias = lax.dot_general(
        b_ref[...],
        ones,
        (((0,), (0,)), ((), ())),
        preferred_element_type=jnp.float32,
    )
    out_ref[...] = acc + bias


def _tc_project(emb2, parity, wT, b2d):
    return pl.pallas_call(
        _mm_body,
        grid=(_N_GRID,),
        in_specs=[
            pl.BlockSpec((B, DPAIR), lambda j: (0, 0)),
            pl.BlockSpec((B, 1), lambda j: (0, 0)),
            pl.BlockSpec((DIM, _N_BLK), lambda j: (0, j)),
            pl.BlockSpec((1, _N_BLK), lambda j: (0, j)),
        ],
        out_specs=pl.BlockSpec((_N_BLK, B), lambda j: (j, 0)),
        out_shape=jax.ShapeDtypeStruct((VOCAB, B), jnp.float32),
    )(emb2, parity, wT, b2d)


@jax.jit
def kernel(inputs_, table, W, b):
    idx = inputs_.astype(jnp.int32)
    t2 = table.reshape(VOCAB // 2, DPAIR)
    emb2 = _make_sc_gather()(idx >> 1, t2)
    parity = (idx & 1).reshape(B, 1)
    out_t = _tc_project(emb2, parity, W.T, b.reshape(1, VOCAB))
    return out_t.T


# speedup vs baseline: 1.0828x; 1.0828x over previous
import functools

import jax
import jax.numpy as jnp
from jax import lax
from jax.experimental import pallas as pl
from jax.experimental.pallas import tpu as pltpu
from jax.experimental.pallas import tpu_sc as plsc

VOCAB = 100000
DIM = 64
DPAD = 128
B = 1024
MAX_NORM = 1.0

_NC = 2
_NS = 16
_NW = _NC * _NS
_B_PER_W = B // _NW

_N_BLK = 2048
_N_GRID = (VOCAB + _N_BLK - 1) // _N_BLK


@functools.lru_cache(maxsize=1)
def _make_sc_gather():
    @functools.partial(
        pl.kernel,
        mesh=plsc.VectorSubcoreMesh(core_axis_name="c", subcore_axis_name="s"),
        out_type=jax.ShapeDtypeStruct((B, DPAD), jnp.float32),
        scratch_types=[
            pltpu.VMEM((_B_PER_W,), jnp.int32),
            pltpu.VMEM((_B_PER_W, DPAD), jnp.float32),
            pltpu.SemaphoreType.DMA,
        ],
    )
    def _sc_gather(idx_hbm, table_hbm, out_hbm, idx_v, rows_v, sem):
        wid = lax.axis_index("s") * _NC + lax.axis_index("c")
        base = wid * _B_PER_W
        pltpu.sync_copy(idx_hbm.at[pl.ds(base, _B_PER_W)], idx_v)
        pltpu.async_copy(table_hbm.at[idx_v], rows_v, sem).wait()
        pltpu.sync_copy(rows_v, out_hbm.at[pl.ds(base, _B_PER_W)])

    return _sc_gather


def _mm_body(emb_ref, wt_ref, b_ref, out_ref, es_ref):
    @pl.when(pl.program_id(0) == 0)
    def _renorm():
        emb = emb_ref[:, :DIM]
        ss = jnp.sum(emb * emb, axis=1, keepdims=True)
        scale = jnp.where(
            ss > MAX_NORM * MAX_NORM, MAX_NORM / (jnp.sqrt(ss) + 1e-7), 1.0
        )
        embs = (emb * scale).astype(jnp.bfloat16)
        ones = jnp.ones((B, 1), jnp.bfloat16)
        zz = jnp.zeros((B, DPAD - DIM - 1), jnp.bfloat16)
        es_ref[...] = jnp.concatenate([embs, ones, zz], axis=1)

    wt = wt_ref[...].astype(jnp.bfloat16)
    bb = b_ref[...].astype(jnp.bfloat16)
    zw = jnp.zeros((DPAD - DIM - 1, _N_BLK), jnp.bfloat16)
    lhs = jnp.concatenate([wt, bb, zw], axis=0)
    out_ref[...] = lax.dot_general(
        lhs,
        es_ref[...],
        (((0,), (1,)), ((), ())),
        preferred_element_type=jnp.float32,
    )


def _tc_project(emb, wT, b2d):
    return pl.pallas_call(
        _mm_body,
        grid=(_N_GRID,),
        in_specs=[
            pl.BlockSpec((B, DPAD), lambda j: (0, 0)),
            pl.BlockSpec((DIM, _N_BLK), lambda j: (0, j)),
            pl.BlockSpec((1, _N_BLK), lambda j: (0, j)),
        ],
        out_specs=pl.BlockSpec((_N_BLK, B), lambda j: (j, 0)),
        out_shape=jax.ShapeDtypeStruct((VOCAB, B), jnp.float32),
        scratch_shapes=[pltpu.VMEM((B, DPAD), jnp.bfloat16)],
    )(emb, wT, b2d)


@jax.jit
def kernel(inputs_, table, W, b):
    idx = inputs_.astype(jnp.int32)
    tpad = jnp.pad(table, ((0, 0), (0, DPAD - DIM)))
    emb = _make_sc_gather()(idx, tpad)
    out_t = _tc_project(emb, W.T, b.reshape(1, VOCAB))
    return out_t.T


# speedup vs baseline: 1.0912x; 1.0077x over previous
import functools

import jax
import jax.numpy as jnp
from jax import lax
from jax.experimental import pallas as pl
from jax.experimental.pallas import tpu as pltpu
from jax.experimental.pallas import tpu_sc as plsc

VOCAB = 100000
DIM = 64
DPAD = 128
B = 1024
MAX_NORM = 1.0

_NC = 2
_NS = 16
_NW = _NC * _NS
_B_PER_W = B // _NW

_N_BLK = 4096
_N_GRID = (VOCAB + _N_BLK - 1) // _N_BLK


@functools.lru_cache(maxsize=1)
def _make_sc_gather():
    @functools.partial(
        pl.kernel,
        mesh=plsc.VectorSubcoreMesh(core_axis_name="c", subcore_axis_name="s"),
        out_type=jax.ShapeDtypeStruct((B, DPAD), jnp.float32),
        scratch_types=[
            pltpu.VMEM((_B_PER_W,), jnp.int32),
            pltpu.VMEM((_B_PER_W, DPAD), jnp.float32),
            pltpu.SemaphoreType.DMA,
        ],
    )
    def _sc_gather(idx_hbm, table_hbm, out_hbm, idx_v, rows_v, sem):
        wid = lax.axis_index("s") * _NC + lax.axis_index("c")
        base = wid * _B_PER_W
        pltpu.sync_copy(idx_hbm.at[pl.ds(base, _B_PER_W)], idx_v)
        pltpu.async_copy(table_hbm.at[idx_v], rows_v, sem).wait()
        pltpu.sync_copy(rows_v, out_hbm.at[pl.ds(base, _B_PER_W)])

    return _sc_gather


def _mm_body(emb_ref, wt_ref, b_ref, out_ref, es_ref):
    @pl.when(pl.program_id(0) == 0)
    def _renorm():
        emb = emb_ref[:, :DIM]
        ss = jnp.sum(emb * emb, axis=1, keepdims=True)
        scale = jnp.where(
            ss > MAX_NORM * MAX_NORM, MAX_NORM / (jnp.sqrt(ss) + 1e-7), 1.0
        )
        embs = (emb * scale).astype(jnp.bfloat16)
        ones = jnp.ones((B, 1), jnp.bfloat16)
        zz = jnp.zeros((B, DPAD - DIM - 1), jnp.bfloat16)
        es_ref[...] = jnp.concatenate([embs, ones, zz], axis=1)

    wt = wt_ref[...].astype(jnp.bfloat16)
    bb = b_ref[...].astype(jnp.bfloat16)
    zw = jnp.zeros((DPAD - DIM - 1, _N_BLK), jnp.bfloat16)
    lhs = jnp.concatenate([wt, bb, zw], axis=0)
    out_ref[...] = lax.dot_general(
        lhs,
        es_ref[...],
        (((0,), (1,)), ((), ())),
        preferred_element_type=jnp.float32,
    )


def _tc_project(emb, wT, b2d):
    return pl.pallas_call(
        _mm_body,
        grid=(_N_GRID,),
        in_specs=[
            pl.BlockSpec((B, DPAD), lambda j: (0, 0)),
            pl.BlockSpec((DIM, _N_BLK), lambda j: (0, j)),
            pl.BlockSpec((1, _N_BLK), lambda j: (0, j)),
        ],
        out_specs=pl.BlockSpec((_N_BLK, B), lambda j: (j, 0)),
        out_shape=jax.ShapeDtypeStruct((VOCAB, B), jnp.float32),
        scratch_shapes=[pltpu.VMEM((B, DPAD), jnp.bfloat16)],
    )(emb, wT, b2d)


@jax.jit
def kernel(inputs_, table, W, b):
    idx = inputs_.astype(jnp.int32)
    tpad = jnp.pad(table, ((0, 0), (0, DPAD - DIM)))
    emb = _make_sc_gather()(idx, tpad)
    out_t = _tc_project(emb, W.T, b.reshape(1, VOCAB))
    return out_t.T


# speedup vs baseline: 1.1306x; 1.0361x over previous
import functools

import jax
import jax.numpy as jnp
from jax import lax
from jax.experimental import pallas as pl
from jax.experimental.pallas import tpu as pltpu
from jax.experimental.pallas import tpu_sc as plsc

VOCAB = 100000
DIM = 64
DPAD = 128
B = 1024
MAX_NORM = 1.0

_NC = 2
_NS = 16
_NW = _NC * _NS
_B_PER_W = B // _NW

_N_BLK = 4096
_N_GRID = (VOCAB + _N_BLK - 1) // _N_BLK


@functools.lru_cache(maxsize=1)
def _make_sc_gather():
    @functools.partial(
        pl.kernel,
        mesh=plsc.VectorSubcoreMesh(core_axis_name="c", subcore_axis_name="s"),
        out_type=jax.ShapeDtypeStruct((B, DPAD), jnp.float32),
        scratch_types=[
            pltpu.VMEM((_B_PER_W,), jnp.int32),
            pltpu.VMEM((_B_PER_W, DPAD), jnp.float32),
            pltpu.SemaphoreType.DMA,
        ],
    )
    def _sc_gather(idx_hbm, table_hbm, out_hbm, idx_v, rows_v, sem):
        wid = lax.axis_index("s") * _NC + lax.axis_index("c")
        base = wid * _B_PER_W
        pltpu.sync_copy(idx_hbm.at[pl.ds(base, _B_PER_W)], idx_v)
        pltpu.async_copy(table_hbm.at[idx_v], rows_v, sem).wait()
        pltpu.sync_copy(rows_v, out_hbm.at[pl.ds(base, _B_PER_W)])

    return _sc_gather


_T_BLK = 2048


def _tp_body(tt_ref, out_ref):
    out_ref[...] = jnp.pad(
        tt_ref[...].T, ((0, 0), (0, DPAD - DIM))
    )


def _transpose_pad(tT):
    return pl.pallas_call(
        _tp_body,
        grid=(VOCAB // _T_BLK + (VOCAB % _T_BLK != 0),),
        in_specs=[pl.BlockSpec((DIM, _T_BLK), lambda j: (0, j))],
        out_specs=pl.BlockSpec((_T_BLK, DPAD), lambda j: (j, 0)),
        out_shape=jax.ShapeDtypeStruct((VOCAB, DPAD), jnp.float32),
    )(tT)


def _mm_body(emb_ref, wt_ref, b_ref, out_ref, es_ref):
    @pl.when(pl.program_id(0) == 0)
    def _renorm():
        emb = emb_ref[:, :DIM]
        ss = jnp.sum(emb * emb, axis=1, keepdims=True)
        scale = jnp.where(
            ss > MAX_NORM * MAX_NORM, MAX_NORM / (jnp.sqrt(ss) + 1e-7), 1.0
        )
        embs = (emb * scale).astype(jnp.bfloat16)
        ones = jnp.ones((B, 1), jnp.bfloat16)
        zz = jnp.zeros((B, DPAD - DIM - 1), jnp.bfloat16)
        es_ref[...] = jnp.concatenate([embs, ones, zz], axis=1)

    wt = wt_ref[...].astype(jnp.bfloat16)
    bb = b_ref[...].astype(jnp.bfloat16)
    zw = jnp.zeros((DPAD - DIM - 1, _N_BLK), jnp.bfloat16)
    lhs = jnp.concatenate([wt, bb, zw], axis=0)
    out_ref[...] = lax.dot_general(
        lhs,
        es_ref[...],
        (((0,), (1,)), ((), ())),
        preferred_element_type=jnp.float32,
    )


def _tc_project(emb, wT, b2d):
    return pl.pallas_call(
        _mm_body,
        grid=(_N_GRID,),
        in_specs=[
            pl.BlockSpec((B, DPAD), lambda j: (0, 0)),
            pl.BlockSpec((DIM, _N_BLK), lambda j: (0, j)),
            pl.BlockSpec((1, _N_BLK), lambda j: (0, j)),
        ],
        out_specs=pl.BlockSpec((_N_BLK, B), lambda j: (j, 0)),
        out_shape=jax.ShapeDtypeStruct((VOCAB, B), jnp.float32),
        scratch_shapes=[pltpu.VMEM((B, DPAD), jnp.bfloat16)],
    )(emb, wT, b2d)


@jax.jit
def kernel(inputs_, table, W, b):
    idx = inputs_.astype(jnp.int32)
    tpad = _transpose_pad(table.T)
    emb = _make_sc_gather()(idx, tpad)
    out_t = _tc_project(emb, W.T, b.reshape(1, VOCAB))
    return out_t.T


# speedup vs baseline: 1.2057x; 1.0664x over previous
import functools

import jax
import jax.numpy as jnp
from jax import lax
from jax.experimental import pallas as pl
from jax.experimental.pallas import tpu as pltpu
from jax.experimental.pallas import tpu_sc as plsc

VOCAB = 100000
DIM = 64
DPAD = 128
B = 1024
MAX_NORM = 1.0

_NC = 2
_NS = 16
_NW = _NC * _NS
_B_PER_W = B // _NW

_N_BLK = 4096
_N_GRID = (VOCAB + _N_BLK - 1) // _N_BLK


@functools.lru_cache(maxsize=1)
def _make_sc_gather():
    @functools.partial(
        pl.kernel,
        mesh=plsc.VectorSubcoreMesh(core_axis_name="c", subcore_axis_name="s"),
        out_type=jax.ShapeDtypeStruct((B, DPAD), jnp.float32),
        scratch_types=[
            pltpu.VMEM((_B_PER_W,), jnp.int32),
            pltpu.VMEM((_B_PER_W, DPAD), jnp.float32),
            pltpu.SemaphoreType.DMA,
        ],
    )
    def _sc_gather(idx_hbm, table_hbm, out_hbm, idx_v, rows_v, sem):
        wid = lax.axis_index("s") * _NC + lax.axis_index("c")
        base = wid * _B_PER_W
        pltpu.sync_copy(idx_hbm.at[pl.ds(base, _B_PER_W)], idx_v)
        pltpu.async_copy(table_hbm.at[idx_v], rows_v, sem).wait()
        pltpu.sync_copy(rows_v, out_hbm.at[pl.ds(base, _B_PER_W)])

    return _sc_gather


_T_BLK = 2048
_HALF = 51200
_T_GRID = _HALF // _T_BLK


def _tp_body(lo_ref, hi_ref, out_ref):
    out_ref[...] = jnp.concatenate([lo_ref[...].T, hi_ref[...].T], axis=1)


def _transpose_pack(tT):
    return pl.pallas_call(
        _tp_body,
        grid=(_T_GRID,),
        in_specs=[
            pl.BlockSpec((DIM, _T_BLK), lambda j: (0, j)),
            pl.BlockSpec(
                (DIM, _T_BLK),
                lambda j: (0, jnp.minimum(j + _T_GRID, VOCAB // _T_BLK)),
            ),
        ],
        out_specs=pl.BlockSpec((_T_BLK, DPAD), lambda j: (j, 0)),
        out_shape=jax.ShapeDtypeStruct((_HALF, DPAD), jnp.float32),
    )(tT, tT)


def _mm_body(emb_ref, par_ref, wt_ref, b_ref, out_ref, es_ref):
    @pl.when(pl.program_id(0) == 0)
    def _renorm():
        emb = jnp.where(
            par_ref[...] > 0, emb_ref[:, DIM:], emb_ref[:, :DIM]
        )
        ss = jnp.sum(emb * emb, axis=1, keepdims=True)
        scale = jnp.where(
            ss > MAX_NORM * MAX_NORM, MAX_NORM / (jnp.sqrt(ss) + 1e-7), 1.0
        )
        embs = (emb * scale).astype(jnp.bfloat16)
        ones = jnp.ones((B, 1), jnp.bfloat16)
        zz = jnp.zeros((B, DPAD - DIM - 1), jnp.bfloat16)
        es_ref[...] = jnp.concatenate([embs, ones, zz], axis=1)

    wt = wt_ref[...].astype(jnp.bfloat16)
    bb = b_ref[...].astype(jnp.bfloat16)
    zw = jnp.zeros((DPAD - DIM - 1, _N_BLK), jnp.bfloat16)
    lhs = jnp.concatenate([wt, bb, zw], axis=0)
    out_ref[...] = lax.dot_general(
        lhs,
        es_ref[...],
        (((0,), (1,)), ((), ())),
        preferred_element_type=jnp.float32,
    )


def _tc_project(emb, parity, wT, b2d):
    return pl.pallas_call(
        _mm_body,
        grid=(_N_GRID,),
        in_specs=[
            pl.BlockSpec((B, DPAD), lambda j: (0, 0)),
            pl.BlockSpec((B, 1), lambda j: (0, 0)),
            pl.BlockSpec((DIM, _N_BLK), lambda j: (0, j)),
            pl.BlockSpec((1, _N_BLK), lambda j: (0, j)),
        ],
        out_specs=pl.BlockSpec((_N_BLK, B), lambda j: (j, 0)),
        out_shape=jax.ShapeDtypeStruct((VOCAB, B), jnp.float32),
        scratch_shapes=[pltpu.VMEM((B, DPAD), jnp.bfloat16)],
    )(emb, parity, wT, b2d)


@jax.jit
def kernel(inputs_, table, W, b):
    idx = inputs_.astype(jnp.int32)
    t2 = _transpose_pack(table.T)
    row = jnp.where(idx < _HALF, idx, idx - _HALF)
    emb2 = _make_sc_gather()(row, t2)
    sel = (idx >= _HALF).astype(jnp.int32).reshape(B, 1)
    out_t = _tc_project(emb2, sel, W.T, b.reshape(1, VOCAB))
    return out_t.T


# speedup vs baseline: 1.2373x; 1.0262x over previous
import functools

import jax
import jax.numpy as jnp
from jax import lax
from jax.experimental import pallas as pl
from jax.experimental.pallas import tpu as pltpu
from jax.experimental.pallas import tpu_sc as plsc

VOCAB = 100000
DIM = 64
DPAD = 128
B = 1024
MAX_NORM = 1.0

_NC = 2
_NS = 16
_NW = _NC * _NS
_B_PER_W = B // _NW

_N_BLK = 4096
_N_GRID = (VOCAB + _N_BLK - 1) // _N_BLK


@functools.lru_cache(maxsize=1)
def _make_sc_gather():
    @functools.partial(
        pl.kernel,
        mesh=plsc.VectorSubcoreMesh(core_axis_name="c", subcore_axis_name="s"),
        out_type=jax.ShapeDtypeStruct((B, DPAD), jnp.float32),
        scratch_types=[
            pltpu.VMEM((_B_PER_W,), jnp.int32),
            pltpu.VMEM((_B_PER_W, DPAD), jnp.float32),
            pltpu.SemaphoreType.DMA,
        ],
    )
    def _sc_gather(idx_hbm, table_hbm, out_hbm, idx_v, rows_v, sem):
        wid = lax.axis_index("s") * _NC + lax.axis_index("c")
        base = wid * _B_PER_W
        pltpu.sync_copy(idx_hbm.at[pl.ds(base, _B_PER_W)], idx_v)
        pltpu.async_copy(table_hbm.at[idx_v], rows_v, sem).wait()
        pltpu.sync_copy(rows_v, out_hbm.at[pl.ds(base, _B_PER_W)])

    return _sc_gather


_T_BLK = 4096
_HALF = 53248
_T_GRID = _HALF // _T_BLK


def _tp_body(lo_ref, hi_ref, out_ref):
    out_ref[...] = jnp.concatenate([lo_ref[...].T, hi_ref[...].T], axis=1)


def _transpose_pack(tT):
    return pl.pallas_call(
        _tp_body,
        grid=(_T_GRID,),
        in_specs=[
            pl.BlockSpec((DIM, _T_BLK), lambda j: (0, j)),
            pl.BlockSpec(
                (DIM, _T_BLK),
                lambda j: (0, jnp.minimum(j + _T_GRID, VOCAB // _T_BLK)),
            ),
        ],
        out_specs=pl.BlockSpec((_T_BLK, DPAD), lambda j: (j, 0)),
        out_shape=jax.ShapeDtypeStruct((_HALF, DPAD), jnp.float32),
    )(tT, tT)


def _mm_body(emb_ref, par_ref, wt_ref, b_ref, out_ref, es_ref):
    @pl.when(pl.program_id(0) == 0)
    def _renorm():
        emb = jnp.where(
            par_ref[...] > 0, emb_ref[:, DIM:], emb_ref[:, :DIM]
        )
        ss = jnp.sum(emb * emb, axis=1, keepdims=True)
        scale = jnp.where(
            ss > MAX_NORM * MAX_NORM, MAX_NORM / (jnp.sqrt(ss) + 1e-7), 1.0
        )
        embs = (emb * scale).astype(jnp.bfloat16)
        ones = jnp.ones((B, 1), jnp.bfloat16)
        zz = jnp.zeros((B, DPAD - DIM - 1), jnp.bfloat16)
        es_ref[...] = jnp.concatenate([embs, ones, zz], axis=1)

    wt = wt_ref[...].astype(jnp.bfloat16)
    bb = b_ref[...].astype(jnp.bfloat16)
    zw = jnp.zeros((DPAD - DIM - 1, _N_BLK), jnp.bfloat16)
    lhs = jnp.concatenate([wt, bb, zw], axis=0)
    out_ref[...] = lax.dot_general(
        lhs,
        es_ref[...],
        (((0,), (1,)), ((), ())),
        preferred_element_type=jnp.float32,
    )


def _tc_project(emb, parity, wT, b2d):
    return pl.pallas_call(
        _mm_body,
        grid=(_N_GRID,),
        in_specs=[
            pl.BlockSpec((B, DPAD), lambda j: (0, 0)),
            pl.BlockSpec((B, 1), lambda j: (0, 0)),
            pl.BlockSpec((DIM, _N_BLK), lambda j: (0, j)),
            pl.BlockSpec((1, _N_BLK), lambda j: (0, j)),
        ],
        out_specs=pl.BlockSpec((_N_BLK, B), lambda j: (j, 0)),
        out_shape=jax.ShapeDtypeStruct((VOCAB, B), jnp.float32),
        scratch_shapes=[pltpu.VMEM((B, DPAD), jnp.bfloat16)],
    )(emb, parity, wT, b2d)


@jax.jit
def kernel(inputs_, table, W, b):
    idx = inputs_.astype(jnp.int32)
    t2 = _transpose_pack(table.T)
    row = jnp.where(idx < _HALF, idx, idx - _HALF)
    emb2 = _make_sc_gather()(row, t2)
    sel = (idx >= _HALF).astype(jnp.int32).reshape(B, 1)
    out_t = _tc_project(emb2, sel, W.T, b.reshape(1, VOCAB))
    return out_t.T


# speedup vs baseline: 1.2457x; 1.0068x over previous
import functools

import jax
import jax.numpy as jnp
from jax import lax
from jax.experimental import pallas as pl
from jax.experimental.pallas import tpu as pltpu
from jax.experimental.pallas import tpu_sc as plsc

VOCAB = 100000
DIM = 64
DPAD = 128
B = 1024
MAX_NORM = 1.0

_NC = 2
_NS = 16
_NW = _NC * _NS
_B_PER_W = B // _NW

_N_BLK = 4096
_N_GRID = (VOCAB + _N_BLK - 1) // _N_BLK


@functools.lru_cache(maxsize=1)
def _make_sc_gather():
    @functools.partial(
        pl.kernel,
        mesh=plsc.VectorSubcoreMesh(core_axis_name="c", subcore_axis_name="s"),
        out_type=jax.ShapeDtypeStruct((B, DPAD), jnp.float32),
        scratch_types=[
            pltpu.VMEM((_B_PER_W,), jnp.int32),
            pltpu.VMEM((_B_PER_W, DPAD), jnp.float32),
            pltpu.SemaphoreType.DMA,
        ],
    )
    def _sc_gather(idx_hbm, table_hbm, out_hbm, idx_v, rows_v, sem):
        wid = lax.axis_index("s") * _NC + lax.axis_index("c")
        base = wid * _B_PER_W
        pltpu.sync_copy(idx_hbm.at[pl.ds(base, _B_PER_W)], idx_v)
        pltpu.async_copy(table_hbm.at[idx_v], rows_v, sem).wait()
        pltpu.sync_copy(rows_v, out_hbm.at[pl.ds(base, _B_PER_W)])

    return _sc_gather


_T_BLK = 8192
_HALF = 57344
_T_GRID = _HALF // _T_BLK


def _tp_body(lo_ref, hi_ref, out_ref):
    out_ref[...] = jnp.concatenate([lo_ref[...].T, hi_ref[...].T], axis=1)


def _transpose_pack(tT):
    return pl.pallas_call(
        _tp_body,
        grid=(_T_GRID,),
        in_specs=[
            pl.BlockSpec((DIM, _T_BLK), lambda j: (0, j)),
            pl.BlockSpec(
                (DIM, _T_BLK),
                lambda j: (0, jnp.minimum(j + _T_GRID, VOCAB // _T_BLK)),
            ),
        ],
        out_specs=pl.BlockSpec((_T_BLK, DPAD), lambda j: (j, 0)),
        out_shape=jax.ShapeDtypeStruct((_HALF, DPAD), jnp.float32),
    )(tT, tT)


def _mm_body(emb_ref, par_ref, wt_ref, b_ref, out_ref, es_ref):
    @pl.when(pl.program_id(0) == 0)
    def _renorm():
        emb = jnp.where(
            par_ref[...] > 0, emb_ref[:, DIM:], emb_ref[:, :DIM]
        )
        ss = jnp.sum(emb * emb, axis=1, keepdims=True)
        scale = jnp.where(
            ss > MAX_NORM * MAX_NORM, MAX_NORM / (jnp.sqrt(ss) + 1e-7), 1.0
        )
        embs = (emb * scale).astype(jnp.bfloat16)
        ones = jnp.ones((B, 1), jnp.bfloat16)
        zz = jnp.zeros((B, DPAD - DIM - 1), jnp.bfloat16)
        es_ref[...] = jnp.concatenate([embs, ones, zz], axis=1)

    wt = wt_ref[...].astype(jnp.bfloat16)
    bb = b_ref[...].astype(jnp.bfloat16)
    zw = jnp.zeros((DPAD - DIM - 1, _N_BLK), jnp.bfloat16)
    lhs = jnp.concatenate([wt, bb, zw], axis=0)
    out_ref[...] = lax.dot_general(
        lhs,
        es_ref[...],
        (((0,), (1,)), ((), ())),
        preferred_element_type=jnp.float32,
    )


def _tc_project(emb, parity, wT, b2d):
    return pl.pallas_call(
        _mm_body,
        grid=(_N_GRID,),
        in_specs=[
            pl.BlockSpec((B, DPAD), lambda j: (0, 0)),
            pl.BlockSpec((B, 1), lambda j: (0, 0)),
            pl.BlockSpec((DIM, _N_BLK), lambda j: (0, j)),
            pl.BlockSpec((1, _N_BLK), lambda j: (0, j)),
        ],
        out_specs=pl.BlockSpec((_N_BLK, B), lambda j: (j, 0)),
        out_shape=jax.ShapeDtypeStruct((VOCAB, B), jnp.float32),
        scratch_shapes=[pltpu.VMEM((B, DPAD), jnp.bfloat16)],
    )(emb, parity, wT, b2d)


@jax.jit
def kernel(inputs_, table, W, b):
    idx = inputs_.astype(jnp.int32)
    t2 = _transpose_pack(table.T)
    row = jnp.where(idx < _HALF, idx, idx - _HALF)
    emb2 = _make_sc_gather()(row, t2)
    sel = (idx >= _HALF).astype(jnp.int32).reshape(B, 1)
    out_t = _tc_project(emb2, sel, W.T, b.reshape(1, VOCAB))
    return out_t.T
